# baseline (speedup 1.0000x reference)
import functools

import jax
import jax.numpy as jnp
import numpy as np
from jax.experimental import pallas as pl
from jax.experimental.pallas import tpu as pltpu

_H = 16
_ST_K = 1024
_ST_V = 2048


def _q_proj_kernel(qg_ref, wq_ref, bq_ref, out_ref):
    out_ref[0] = jax.lax.dot_general(
        qg_ref[0], wq_ref[...], (((1,), (1,)), ((), ())),
        preferred_element_type=jnp.float32) + bq_ref[...]


def _k_proj_kernel(k_ref, wk_ref, kt_ref):
    kt_ref[0] = jax.lax.dot_general(
        wk_ref[...], k_ref[0], (((1,), (1,)), ((), ())),
        preferred_element_type=jnp.float32)


def _attn_kernel(u, up, s_len, q_ref, kt_ref, attn_ref):
    q = q_ref[0, 0]
    kt = kt_ref[0]
    qs = q[:up]
    qf = q[up:]

    ss = jnp.dot(qs, kt, preferred_element_type=jnp.float32)
    m_col = (jnp.max(ss, axis=1, keepdims=True)
             - jnp.sum(ss, axis=1, keepdims=True) / s_len)
    rows1 = jax.lax.broadcasted_iota(jnp.int32, (up, 1), 0)
    m_col = jnp.where(rows1 < u, m_col, jnp.float32(-1e30))
    m_row = jnp.transpose(m_col)

    rows = jax.lax.broadcasted_iota(jnp.int32, (up, up), 0)
    cols = jax.lax.broadcasted_iota(jnp.int32, (up, up), 1)
    beats = (m_row > m_col) | ((m_row == m_col) & (cols < rows))
    ranks = jnp.sum(beats.astype(jnp.int32), axis=1, keepdims=True)
    perm_mat = (rows == jnp.transpose(ranks)).astype(jnp.float32)

    fs = jnp.dot(qf, kt, preferred_element_type=jnp.float32)
    fs_max = jnp.max(fs, axis=1, keepdims=True)
    ex = jnp.exp(fs - fs_max)
    attn_f = ex / jnp.sum(ex, axis=1, keepdims=True)
    attn_p = jnp.dot(perm_mat, attn_f, preferred_element_type=jnp.float32)
    attn_ref[0, 0] = attn_p[:u]


def _ctx_kernel(u, n_vt, attn_ref, v_ref, wv_ref, wo_ref, bv_ref, bo_ref,
                out_ref, u_s):
    h_all = _H
    dh = wv_ref.shape[0] // h_all
    t = pl.program_id(1)
    vblk = v_ref[0]

    for h in range(h_all):
        part = jnp.dot(attn_ref[0, h], vblk,
                       preferred_element_type=jnp.float32)

        @pl.when(t == 0)
        def _init():
            u_s[h, :u] = part

        @pl.when(t > 0)
        def _add():
            u_s[h, :u] += part

    @pl.when(t == n_vt - 1)
    def _finish():
        acc = jax.lax.dot_general(
            bv_ref[...], wo_ref[...], (((1,), (1,)), ((), ())),
            preferred_element_type=jnp.float32) + bo_ref[...]
        acc = jnp.broadcast_to(acc, (u, wo_ref.shape[0]))
        for h in range(h_all):
            ctx_h = jax.lax.dot_general(
                u_s[h, :u], wv_ref[h * dh:(h + 1) * dh, :],
                (((1,), (1,)), ((), ())),
                preferred_element_type=jnp.float32)
            acc = acc + jnp.dot(
                ctx_h, jnp.transpose(wo_ref[:, h * dh:(h + 1) * dh]),
                preferred_element_type=jnp.float32)
        out_ref[0] = acc


def kernel(queries, keys, values, Wq, bq, Wk, bk, Wv, bv, Wo, bo):
    del bk
    B, L, D = queries.shape
    S = keys.shape[1]
    H = _H
    dh = D // H
    u = min(5 * int(np.ceil(np.log(S))), L)
    up = (u + 7) // 8 * 8
    n_vt = S // _ST_V

    perm = jax.random.permutation(jax.random.key(42), L)[:u]
    pad = ((0, 0), (0, up - u), (0, 0))
    qg = jnp.concatenate([
        jnp.pad(queries[:, perm, :], pad),
        jnp.pad(queries[:, :u, :], pad),
    ], axis=1)

    q_proj = pl.pallas_call(
        _q_proj_kernel,
        grid=(B,),
        in_specs=[
            pl.BlockSpec((1, 2 * up, D), lambda b: (b, 0, 0)),
            pl.BlockSpec((D, D), lambda b: (0, 0)),
            pl.BlockSpec((1, D), lambda b: (0, 0)),
        ],
        out_specs=pl.BlockSpec((1, 2 * up, D), lambda b: (b, 0, 0)),
        out_shape=jax.ShapeDtypeStruct((B, 2 * up, D), jnp.float32),
        compiler_params=pltpu.CompilerParams(
            dimension_semantics=("parallel",)),
    )(qg, Wq, bq.reshape(1, D))
    q_heads = q_proj.reshape(B, 2 * up, H, dh).transpose(0, 2, 1, 3)

    kt = pl.pallas_call(
        _k_proj_kernel,
        grid=(B, S // _ST_K),
        in_specs=[
            pl.BlockSpec((1, _ST_K, D), lambda b, s: (b, s, 0)),
            pl.BlockSpec((D, D), lambda b, s: (0, 0)),
        ],
        out_specs=pl.BlockSpec((1, D, _ST_K), lambda b, s: (b, 0, s)),
        out_shape=jax.ShapeDtypeStruct((B, D, S), jnp.float32),
        compiler_params=pltpu.CompilerParams(
            dimension_semantics=("parallel", "parallel")),
    )(keys, Wk)

    attn = pl.pallas_call(
        functools.partial(_attn_kernel, u, up, S),
        grid=(B, H),
        in_specs=[
            pl.BlockSpec((1, 1, 2 * up, dh), lambda b, h: (b, h, 0, 0)),
            pl.BlockSpec((1, dh, S), lambda b, h: (b, h, 0)),
        ],
        out_specs=pl.BlockSpec((1, 1, u, S), lambda b, h: (b, h, 0, 0)),
        out_shape=jax.ShapeDtypeStruct((B, H, u, S), jnp.float32),
        compiler_params=pltpu.CompilerParams(
            dimension_semantics=("parallel", "parallel")),
    )(q_heads, kt)

    out = pl.pallas_call(
        functools.partial(_ctx_kernel, u, n_vt),
        grid=(B, n_vt),
        in_specs=[
            pl.BlockSpec((1, H, u, _ST_V), lambda b, t: (b, 0, 0, t)),
            pl.BlockSpec((1, _ST_V, D), lambda b, t: (b, t, 0)),
            pl.BlockSpec((D, D), lambda b, t: (0, 0)),
            pl.BlockSpec((D, D), lambda b, t: (0, 0)),
            pl.BlockSpec((1, D), lambda b, t: (0, 0)),
            pl.BlockSpec((1, D), lambda b, t: (0, 0)),
        ],
        out_specs=pl.BlockSpec((1, u, D), lambda b, t: (b, 0, 0)),
        out_shape=jax.ShapeDtypeStruct((B, u, D), jnp.float32),
        scratch_shapes=[pltpu.VMEM((H, up, D), jnp.float32)],
        compiler_params=pltpu.CompilerParams(
            dimension_semantics=("parallel", "arbitrary")),
    )(attn, values, Wv, Wo, bv.reshape(1, D), bo.reshape(1, D))

    return (out, attn)


# speedup vs baseline: 1.2665x; 1.2665x over previous
import functools

import jax
import jax.numpy as jnp
import numpy as np
from jax.experimental import pallas as pl
from jax.experimental.pallas import tpu as pltpu

_H = 16


def _q_proj_kernel(qg_ref, wq_ref, bq_ref, out_ref):
    out_ref[0] = jax.lax.dot_general(
        qg_ref[0], wq_ref[...], (((1,), (1,)), ((), ())),
        preferred_element_type=jnp.float32) + bq_ref[...]


def _kv_proj_kernel(k_ref, v_ref, wk_ref, wv_ref, kt_ref, vt_ref):
    kt_ref[0] = jax.lax.dot_general(
        wk_ref[...], k_ref[0], (((1,), (1,)), ((), ())),
        preferred_element_type=jnp.float32)
    vt_ref[0] = jax.lax.dot_general(
        wv_ref[...].astype(jnp.bfloat16), v_ref[0].astype(jnp.bfloat16),
        (((1,), (1,)), ((), ())),
        preferred_element_type=jnp.float32).astype(jnp.bfloat16)


def _attn_kernel(u, up, s_len, q_ref, kt_ref, vt_ref, attn_ref, ctx_ref):
    q = q_ref[0, 0]
    kt = kt_ref[0]
    vt = vt_ref[0]
    qs = q[:up]
    qf = q[up:]

    ss = jnp.dot(qs, kt, preferred_element_type=jnp.float32)
    m_col = (jnp.max(ss, axis=1, keepdims=True)
             - jnp.sum(ss, axis=1, keepdims=True) / s_len)
    rows1 = jax.lax.broadcasted_iota(jnp.int32, (up, 1), 0)
    m_col = jnp.where(rows1 < u, m_col, jnp.float32(-1e30))
    m_row = jnp.transpose(m_col)

    rows = jax.lax.broadcasted_iota(jnp.int32, (up, up), 0)
    cols = jax.lax.broadcasted_iota(jnp.int32, (up, up), 1)
    beats = (m_row > m_col) | ((m_row == m_col) & (cols < rows))
    ranks = jnp.sum(beats.astype(jnp.int32), axis=1, keepdims=True)
    perm_mat = (rows == jnp.transpose(ranks)).astype(jnp.float32)

    fs = jnp.dot(qf, kt, preferred_element_type=jnp.float32)
    fs_max = jnp.max(fs, axis=1, keepdims=True)
    ex = jnp.exp(fs - fs_max)
    attn_f = ex / jnp.sum(ex, axis=1, keepdims=True)
    attn_p = jnp.dot(perm_mat, attn_f, preferred_element_type=jnp.float32)
    attn_ref[0, 0] = attn_p[:u]

    ctx_ref[0, 0] = jax.lax.dot_general(
        attn_p.astype(jnp.bfloat16), vt, (((1,), (1,)), ((), ())),
        preferred_element_type=jnp.float32)


def _out_proj_kernel(ctx_ref, wo_ref, bv_ref, bo_ref, out_ref):
    out_ref[0] = jax.lax.dot_general(
        ctx_ref[0] + bv_ref[...], wo_ref[...], (((1,), (1,)), ((), ())),
        preferred_element_type=jnp.float32) + bo_ref[...]


def kernel(queries, keys, values, Wq, bq, Wk, bk, Wv, bv, Wo, bo):
    del bk
    B, L, D = queries.shape
    S = keys.shape[1]
    H = _H
    dh = D // H
    u = min(5 * int(np.ceil(np.log(S))), L)
    up = (u + 7) // 8 * 8

    perm = jax.random.permutation(jax.random.key(42), L)[:u]
    pad = ((0, 0), (0, up - u), (0, 0))
    qg = jnp.concatenate([
        jnp.pad(queries[:, perm, :], pad),
        jnp.pad(queries[:, :u, :], pad),
    ], axis=1)

    q_proj = pl.pallas_call(
        _q_proj_kernel,
        grid=(B,),
        in_specs=[
            pl.BlockSpec((1, 2 * up, D), lambda b: (b, 0, 0)),
            pl.BlockSpec((D, D), lambda b: (0, 0)),
            pl.BlockSpec((1, D), lambda b: (0, 0)),
        ],
        out_specs=pl.BlockSpec((1, 2 * up, D), lambda b: (b, 0, 0)),
        out_shape=jax.ShapeDtypeStruct((B, 2 * up, D), jnp.float32),
        compiler_params=pltpu.CompilerParams(
            dimension_semantics=("parallel",)),
    )(qg, Wq, bq.reshape(1, D))
    q_heads = q_proj.reshape(B, 2 * up, H, dh).transpose(0, 2, 1, 3)

    ST = 512
    kt, vt = pl.pallas_call(
        _kv_proj_kernel,
        grid=(B, S // ST),
        in_specs=[
            pl.BlockSpec((1, ST, D), lambda b, s: (b, s, 0)),
            pl.BlockSpec((1, ST, D), lambda b, s: (b, s, 0)),
            pl.BlockSpec((D, D), lambda b, s: (0, 0)),
            pl.BlockSpec((D, D), lambda b, s: (0, 0)),
        ],
        out_specs=[
            pl.BlockSpec((1, D, ST), lambda b, s: (b, 0, s)),
            pl.BlockSpec((1, D, ST), lambda b, s: (b, 0, s)),
        ],
        out_shape=[
            jax.ShapeDtypeStruct((B, D, S), jnp.float32),
            jax.ShapeDtypeStruct((B, D, S), jnp.bfloat16),
        ],
        compiler_params=pltpu.CompilerParams(
            dimension_semantics=("parallel", "parallel")),
    )(keys, values, Wk, Wv)

    attn, ctx = pl.pallas_call(
        functools.partial(_attn_kernel, u, up, S),
        grid=(B, H),
        in_specs=[
            pl.BlockSpec((1, 1, 2 * up, dh), lambda b, h: (b, h, 0, 0)),
            pl.BlockSpec((1, dh, S), lambda b, h: (b, h, 0)),
            pl.BlockSpec((1, dh, S), lambda b, h: (b, h, 0)),
        ],
        out_specs=[
            pl.BlockSpec((1, 1, u, S), lambda b, h: (b, h, 0, 0)),
            pl.BlockSpec((1, 1, up, dh), lambda b, h: (b, h, 0, 0)),
        ],
        out_shape=[
            jax.ShapeDtypeStruct((B, H, u, S), jnp.float32),
            jax.ShapeDtypeStruct((B, H, up, dh), jnp.float32),
        ],
        compiler_params=pltpu.CompilerParams(
            dimension_semantics=("parallel", "parallel")),
    )(q_heads, kt, vt)

    ctx_all = ctx.transpose(0, 2, 1, 3).reshape(B, up, D)
    out = pl.pallas_call(
        _out_proj_kernel,
        grid=(B,),
        in_specs=[
            pl.BlockSpec((1, up, D), lambda b: (b, 0, 0)),
            pl.BlockSpec((D, D), lambda b: (0, 0)),
            pl.BlockSpec((1, D), lambda b: (0, 0)),
            pl.BlockSpec((1, D), lambda b: (0, 0)),
        ],
        out_specs=pl.BlockSpec((1, up, D), lambda b: (b, 0, 0)),
        out_shape=jax.ShapeDtypeStruct((B, up, D), jnp.float32),
        compiler_params=pltpu.CompilerParams(
            dimension_semantics=("parallel",)),
    )(ctx_all, Wo, bv.reshape(1, D), bo.reshape(1, D))

    return (out[:, :u, :], attn)


# speedup vs baseline: 1.3086x; 1.0332x over previous
import functools

import jax
import jax.numpy as jnp
import numpy as np
from jax.experimental import pallas as pl
from jax.experimental.pallas import tpu as pltpu

_H = 16


def _q_proj_kernel(qg_ref, wq_ref, bq_ref, out_ref):
    out_ref[0] = jax.lax.dot_general(
        qg_ref[0], wq_ref[...], (((1,), (1,)), ((), ())),
        preferred_element_type=jnp.float32) + bq_ref[...]


def _kv_proj_kernel(k_ref, v_ref, wk_ref, wv_ref, kt_ref, vt_ref):
    kt_ref[0] = jax.lax.dot_general(
        wk_ref[...], k_ref[0], (((1,), (1,)), ((), ())),
        preferred_element_type=jnp.float32)
    vt_ref[0] = jax.lax.dot_general(
        wv_ref[...].astype(jnp.bfloat16), v_ref[0].astype(jnp.bfloat16),
        (((1,), (1,)), ((), ())),
        preferred_element_type=jnp.float32).astype(jnp.bfloat16)


def _attn_kernel(u, up, s_len, q_ref, kt_ref, vt_ref, attn_ref, ctx_ref):
    q = q_ref[0, 0]
    kt = kt_ref[0]
    vt = vt_ref[0]
    qs = q[:up]
    qf = q[up:]

    ss = jnp.dot(qs, kt, preferred_element_type=jnp.float32)
    m_col = (jnp.max(ss, axis=1, keepdims=True)
             - jnp.sum(ss, axis=1, keepdims=True) / s_len)
    rows1 = jax.lax.broadcasted_iota(jnp.int32, (up, 1), 0)
    m_col = jnp.where(rows1 < u, m_col, jnp.float32(-1e30))
    m_row = jnp.transpose(m_col)

    rows = jax.lax.broadcasted_iota(jnp.int32, (up, up), 0)
    cols = jax.lax.broadcasted_iota(jnp.int32, (up, up), 1)
    beats = (m_row > m_col) | ((m_row == m_col) & (cols < rows))
    ranks = jnp.sum(beats.astype(jnp.int32), axis=1, keepdims=True)
    perm_mat = (rows == jnp.transpose(ranks)).astype(jnp.float32)

    fs = jnp.dot(qf, kt, preferred_element_type=jnp.float32)
    fs_max = jnp.max(fs, axis=1, keepdims=True)
    ex = jnp.exp(fs - fs_max)
    attn_f = ex / jnp.sum(ex, axis=1, keepdims=True)
    attn_p = jnp.dot(perm_mat, attn_f, preferred_element_type=jnp.float32)
    attn_ref[0, 0] = attn_p[:u]

    ctx_ref[0, 0] = jax.lax.dot_general(
        attn_p.astype(jnp.bfloat16), vt, (((1,), (1,)), ((), ())),
        preferred_element_type=jnp.float32)


def _out_proj_kernel(ctx_ref, wo_ref, bv_ref, bo_ref, out_ref):
    out_ref[0] = jax.lax.dot_general(
        ctx_ref[0] + bv_ref[...], wo_ref[...], (((1,), (1,)), ((), ())),
        preferred_element_type=jnp.float32) + bo_ref[...]


def kernel(queries, keys, values, Wq, bq, Wk, bk, Wv, bv, Wo, bo):
    del bk
    B, L, D = queries.shape
    S = keys.shape[1]
    H = _H
    dh = D // H
    u = min(5 * int(np.ceil(np.log(S))), L)
    up = (u + 7) // 8 * 8

    perm = jax.random.permutation(jax.random.key(42), L)[:u]
    pad = ((0, 0), (0, up - u), (0, 0))
    qg = jnp.concatenate([
        jnp.pad(queries[:, perm, :], pad),
        jnp.pad(queries[:, :u, :], pad),
    ], axis=1)

    q_proj = pl.pallas_call(
        _q_proj_kernel,
        grid=(B,),
        in_specs=[
            pl.BlockSpec((1, 2 * up, D), lambda b: (b, 0, 0)),
            pl.BlockSpec((D, D), lambda b: (0, 0)),
            pl.BlockSpec((1, D), lambda b: (0, 0)),
        ],
        out_specs=pl.BlockSpec((1, 2 * up, D), lambda b: (b, 0, 0)),
        out_shape=jax.ShapeDtypeStruct((B, 2 * up, D), jnp.float32),
        compiler_params=pltpu.CompilerParams(
            dimension_semantics=("parallel",)),
    )(qg, Wq, bq.reshape(1, D))
    q_heads = q_proj.reshape(B, 2 * up, H, dh).transpose(0, 2, 1, 3)

    ST = 1024
    kt, vt = pl.pallas_call(
        _kv_proj_kernel,
        grid=(B, S // ST),
        in_specs=[
            pl.BlockSpec((1, ST, D), lambda b, s: (b, s, 0)),
            pl.BlockSpec((1, ST, D), lambda b, s: (b, s, 0)),
            pl.BlockSpec((D, D), lambda b, s: (0, 0)),
            pl.BlockSpec((D, D), lambda b, s: (0, 0)),
        ],
        out_specs=[
            pl.BlockSpec((1, D, ST), lambda b, s: (b, 0, s)),
            pl.BlockSpec((1, D, ST), lambda b, s: (b, 0, s)),
        ],
        out_shape=[
            jax.ShapeDtypeStruct((B, D, S), jnp.float32),
            jax.ShapeDtypeStruct((B, D, S), jnp.bfloat16),
        ],
        compiler_params=pltpu.CompilerParams(
            dimension_semantics=("parallel", "parallel")),
    )(keys, values, Wk, Wv)

    attn, ctx = pl.pallas_call(
        functools.partial(_attn_kernel, u, up, S),
        grid=(B, H),
        in_specs=[
            pl.BlockSpec((1, 1, 2 * up, dh), lambda b, h: (b, h, 0, 0)),
            pl.BlockSpec((1, dh, S), lambda b, h: (b, h, 0)),
            pl.BlockSpec((1, dh, S), lambda b, h: (b, h, 0)),
        ],
        out_specs=[
            pl.BlockSpec((1, 1, u, S), lambda b, h: (b, h, 0, 0)),
            pl.BlockSpec((1, 1, up, dh), lambda b, h: (b, h, 0, 0)),
        ],
        out_shape=[
            jax.ShapeDtypeStruct((B, H, u, S), jnp.float32),
            jax.ShapeDtypeStruct((B, H, up, dh), jnp.float32),
        ],
        compiler_params=pltpu.CompilerParams(
            dimension_semantics=("parallel", "parallel")),
    )(q_heads, kt, vt)

    ctx_all = ctx.transpose(0, 2, 1, 3).reshape(B, up, D)
    out = pl.pallas_call(
        _out_proj_kernel,
        grid=(B,),
        in_specs=[
            pl.BlockSpec((1, up, D), lambda b: (b, 0, 0)),
            pl.BlockSpec((D, D), lambda b: (0, 0)),
            pl.BlockSpec((1, D), lambda b: (0, 0)),
            pl.BlockSpec((1, D), lambda b: (0, 0)),
        ],
        out_specs=pl.BlockSpec((1, up, D), lambda b: (b, 0, 0)),
        out_shape=jax.ShapeDtypeStruct((B, up, D), jnp.float32),
        compiler_params=pltpu.CompilerParams(
            dimension_semantics=("parallel",)),
    )(ctx_all, Wo, bv.reshape(1, D), bo.reshape(1, D))

    return (out[:, :u, :], attn)
